# Initial kernel scaffold; baseline (speedup 1.0000x reference)
import jax
import jax.numpy as jnp
from jax.experimental import pallas as pl


def kernel(region_attributes, distance, edge_index, W0, al0, ar0, b0, W1, al1, ar1, b1, W2, al2, ar2, b2, resW2, lam1, lam2, lam3, beta, G):
    raise NotImplementedError("write your pallas kernel here")


# speedup vs baseline: 24.8960x; 24.8960x over previous
import functools
import jax
import jax.numpy as jnp
from jax import lax
from jax.experimental import pallas as pl
from jax.experimental.pallas import tpu as pltpu
from jax.experimental.pallas import tpu_sc as plsc

NN = 2048
EE = 32768
NEG = 0.2
EW = 16
CH = 128
NWORK = 32
CPW = EE // (NWORK * CH)


def _head_masks(fin, heads, hd):
    c = lax.broadcasted_iota(jnp.int32, (fin, EW), 0) // hd
    j = lax.broadcasted_iota(jnp.int32, (fin, EW), 1)
    ml = (c == j).astype(jnp.float32) * (j < heads).astype(jnp.float32)
    mr = (c == (j - 4)).astype(jnp.float32) * ((j - 4) >= 0).astype(jnp.float32) \
        * ((j - 4) < heads).astype(jnp.float32)
    return ml, mr


def _sentinel(heads):
    lane = lax.broadcasted_iota(jnp.int32, (NN, EW), 1)
    return (lane < heads).astype(jnp.float32)


def _prep0_body(x_ref, w_ref, al_ref, ar_ref, feat_ref, eler_ref):
    feat = jnp.dot(x_ref[...], w_ref[...], preferred_element_type=jnp.float32)
    feat_ref[...] = jnp.concatenate([feat, _sentinel(4)], axis=1)
    ml, mr = _head_masks(256, 4, 64)
    eler_ref[...] = (
        jnp.dot(feat * al_ref[...], ml, preferred_element_type=jnp.float32)
        + jnp.dot(feat * ar_ref[...], mr, preferred_element_type=jnp.float32))


def _tc_prep0(x, w0, al_row, ar_row):
    return pl.pallas_call(
        _prep0_body,
        out_shape=(jax.ShapeDtypeStruct((NN, 256 + EW), jnp.float32),
                   jax.ShapeDtypeStruct((NN, EW), jnp.float32)),
    )(x, w0, al_row, ar_row)


def _esum_bcast4(es):
    return jnp.concatenate(
        [jnp.broadcast_to(es[:, h:h + 1], (NN, 64)) for h in range(4)], axis=1)


def _post_prep_body(acc_ref, res_ref, b_ref, w_ref, al_ref, ar_ref,
                    h_ref, feat_ref, eler_ref, *, identity_res, fout, heads,
                    hd, next_heads):
    acc = acc_ref[0] + acc_ref[1]
    es = acc[:, 256:256 + 4]
    rst = acc[:, 0:256] / (_esum_bcast4(es) + 1e-9)
    if identity_res:
        rst = rst + res_ref[...]
    rst = rst + b_ref[...]
    h = jnp.where(rst > 0, rst, jnp.exp(rst) - 1.0)
    h_ref[...] = h
    feat = jnp.dot(h, w_ref[...], preferred_element_type=jnp.float32)
    feat_ref[...] = jnp.concatenate([feat, _sentinel(next_heads)], axis=1)
    ml, mr = _head_masks(fout, heads, hd)
    eler_ref[...] = (
        jnp.dot(feat * al_ref[...], ml, preferred_element_type=jnp.float32)
        + jnp.dot(feat * ar_ref[...], mr, preferred_element_type=jnp.float32))


def _tc_post_prep(acc, res, b_row, w, al_row, ar_row, identity_res, fout,
                  heads, hd, next_heads):
    body = functools.partial(_post_prep_body, identity_res=identity_res,
                             fout=fout, heads=heads, hd=hd,
                             next_heads=next_heads)
    return pl.pallas_call(
        body,
        out_shape=(jax.ShapeDtypeStruct((NN, 256), jnp.float32),
                   jax.ShapeDtypeStruct((NN, fout + EW), jnp.float32),
                   jax.ShapeDtypeStruct((NN, EW), jnp.float32)),
    )(acc, res, b_row, w, al_row, ar_row)


def _emb_body(acc_ref, h_ref, rw_ref, b_ref, emb_ref):
    acc = acc_ref[0] + acc_ref[1]
    es = acc[:, EW:EW + 1]
    emb_ref[...] = (acc[:, 0:EW] / (es + 1e-9)
                    + jnp.dot(h_ref[...], rw_ref[...],
                              preferred_element_type=jnp.float32)
                    + b_ref[...])


def _tc_emb(acc, h2, rw_pad, b_row):
    return pl.pallas_call(
        _emb_body,
        out_shape=jax.ShapeDtypeStruct((NN, EW), jnp.float32),
    )(acc, h2, rw_pad, b_row)


def _grav_body(emb_ref, embt_ref, dist_ref, scal_ref, out_ref):
    i0 = pl.program_id(0) * 128
    acc = jnp.zeros((128, NN), jnp.float32)
    for k in range(1, 5):
        d = embt_ref[k:k + 1, :] - emb_ref[:, k:k + 1]
        acc = acc + d * d
    lam1 = scal_ref[0]
    lam2 = scal_ref[1]
    lam3 = scal_ref[2]
    beta = scal_ref[3]
    g = scal_ref[4]
    r = jnp.sqrt(jnp.maximum(acc * 0.25 + beta * dist_ref[...], 1e-7))
    m1 = jnp.abs(embt_ref[0:1, :])
    m2 = jnp.abs(emb_ref[:, 0:1])
    od = g * jnp.power(m1, lam1) * jnp.power(m2, lam2) / jnp.power(r, lam3)
    row = lax.broadcasted_iota(jnp.int32, (128, NN), 0) + i0
    col = lax.broadcasted_iota(jnp.int32, (128, NN), 1)
    out_ref[...] = jnp.where(row == col, 0.0, od)


def _tc_grav(emb, embt, distance, scal):
    return pl.pallas_call(
        _grav_body,
        grid=(NN // 128,),
        in_specs=[
            pl.BlockSpec((128, EW), lambda i: (i, 0)),
            pl.BlockSpec((EW, NN), lambda i: (0, 0)),
            pl.BlockSpec((128, NN), lambda i: (i, 0)),
            pl.BlockSpec(memory_space=pltpu.SMEM),
        ],
        out_specs=pl.BlockSpec((128, NN), lambda i: (i, 0)),
        out_shape=jax.ShapeDtypeStruct((NN, NN), jnp.float32),
    )(emb, embt, distance, scal)


def _edge_sc_call(fw, heads):
    hseg = (fw - EW) // (heads * 16)
    mesh = plsc.VectorSubcoreMesh(core_axis_name="c", subcore_axis_name="s")
    out_type = jax.ShapeDtypeStruct((2, NN, fw), jnp.float32)
    scratch = [
        pltpu.VMEM((CPW, CH), jnp.int32),
        pltpu.VMEM((CPW, CH), jnp.int32),
        pltpu.VMEM((NN * EW,), jnp.float32),
        pltpu.VMEM((CH, fw), jnp.float32),
        pltpu.VMEM((CH * EW,), jnp.float32),
        pltpu.VMEM_SHARED((NN, fw), jnp.float32),
        pltpu.SemaphoreType.DMA,
    ]

    @functools.partial(
        pl.kernel, mesh=mesh, out_type=out_type, scratch_types=scratch,
        compiler_params=pltpu.CompilerParams(needs_layout_passes=False,
                                             use_tc_tiling_on_sc=False))
    def k(src_hbm, dst_hbm, eler_hbm, feat_hbm, acc_out,
          src_v, dst_v, eler_v, rows_v, w_v, acc_sp, sem):
        cid = lax.axis_index("c")
        sid = lax.axis_index("s")
        wid = cid * 16 + sid
        pltpu.sync_copy(src_hbm.at[pl.ds(wid * CPW, CPW)], src_v)
        pltpu.sync_copy(dst_hbm.at[pl.ds(wid * CPW, CPW)], dst_v)
        pltpu.sync_copy(eler_hbm, eler_v)

        zero16 = jnp.zeros((16,), jnp.float32)

        def zboth(i, _):
            w_v[pl.ds(i * EW, 16)] = zero16
            for s in range(fw // 16):
                rows_v[i, pl.ds(s * 16, 16)] = zero16
            return 0

        lax.fori_loop(0, CH, zboth, 0)
        pltpu.sync_copy(rows_v, acc_sp.at[pl.ds(sid * 128, 128)])
        plsc.subcore_barrier()

        ones16 = jnp.full((16,), 1, jnp.int32)

        for c in range(CPW):
            cp = pltpu.async_copy(feat_hbm.at[src_v.at[c]], rows_v, sem)
            for g in range(CH // 16):
                sidx = src_v[c, pl.ds(g * 16, 16)]
                didx = dst_v[c, pl.ds(g * 16, 16)]
                rowi = (lax.iota(jnp.int32, 16) + g * 16) * EW
                for h in range(heads):
                    el = plsc.load_gather(eler_v, [sidx * EW + h])
                    er = plsc.load_gather(eler_v, [didx * EW + (4 + h)])
                    e = el + er
                    w = jnp.exp(jnp.maximum(e, NEG * e))
                    plsc.store_scatter(w_v, [rowi + h], w)
            cp.wait()

            def scale_edge(i, _):
                wvec = w_v[pl.ds(i * EW, 16)]
                for h in range(heads):
                    wh = plsc.load_gather(w_v, [ones16 * (i * EW + h)])
                    for s in range(hseg):
                        sl = pl.ds((h * hseg + s) * 16, 16)
                        rows_v[i, sl] = rows_v[i, sl] * wh
                sl = pl.ds(fw - EW, 16)
                rows_v[i, sl] = rows_v[i, sl] * wvec
                return 0

            lax.fori_loop(0, CH, scale_edge, 0)
            pltpu.sync_copy(rows_v, acc_sp.at[dst_v.at[c]], add=True)

        plsc.subcore_barrier()
        pltpu.sync_copy(acc_sp.at[pl.ds(sid * 128, 128)], rows_v)
        pltpu.sync_copy(rows_v, acc_out.at[cid, pl.ds(sid * 128, 128)])

    return k


def _edge_pass(src2d, dst2d, eler_flat, feat, fw, heads):
    return _edge_sc_call(fw, heads)(src2d, dst2d, eler_flat, feat)


def kernel(region_attributes, distance, edge_index, W0, al0, ar0, b0,
           W1, al1, ar1, b1, W2, al2, ar2, b2, resW2,
           lam1, lam2, lam3, beta, G):
    x = region_attributes
    src2d = edge_index[0].astype(jnp.int32).reshape(EE // CH, CH)
    dst2d = edge_index[1].astype(jnp.int32).reshape(EE // CH, CH)

    al0r = al0.reshape(1, 256)
    ar0r = ar0.reshape(1, 256)
    al1r = al1.reshape(1, 256)
    ar1r = ar1.reshape(1, 256)
    pad5 = lambda a: jnp.pad(a.reshape(1, 5), ((0, 0), (0, EW - 5)))
    al2r = pad5(al2)
    ar2r = pad5(ar2)
    b2r = pad5(b2)
    w2p = jnp.pad(W2, ((0, 0), (0, EW - 5)))
    rw2p = jnp.pad(resW2, ((0, 0), (0, EW - 5)))

    feat0, eler0 = _tc_prep0(x, W0, al0r, ar0r)
    acc0 = _edge_pass(src2d, dst2d, eler0.reshape(-1), feat0, 256 + EW, 4)
    h1, feat1, eler1 = _tc_post_prep(acc0, feat0[:, 0:256], b0.reshape(1, 256),
                                     W1, al1r, ar1r, False, 256, 4, 64, 4)
    acc1 = _edge_pass(src2d, dst2d, eler1.reshape(-1), feat1, 256 + EW, 4)
    h2, feat2, eler2 = _tc_post_prep(acc1, h1, b1.reshape(1, 256),
                                     w2p, al2r, ar2r, True, EW, 1, 5, 1)
    acc2 = _edge_pass(src2d, dst2d, eler2.reshape(-1), feat2, EW + EW, 1)
    emb = _tc_emb(acc2, h2, rw2p, b2r)
    embt = emb.T
    scal = jnp.concatenate([lam1, lam2, lam3, beta, G]).astype(jnp.float32)
    return _tc_grav(emb, embt, distance, scal)


# speedup vs baseline: 32.7239x; 1.3144x over previous
import functools
import jax
import jax.numpy as jnp
from jax import lax
from jax.experimental import pallas as pl
from jax.experimental.pallas import tpu as pltpu
from jax.experimental.pallas import tpu_sc as plsc

NN = 2048
EE = 32768
NEG = 0.2
EW = 16
CH = 64
NWORK = 32
NCH = EE // (NWORK * CH)
NBUF = 3


def _head_masks(fin, heads, hd):
    c = lax.broadcasted_iota(jnp.int32, (fin, EW), 0) // hd
    j = lax.broadcasted_iota(jnp.int32, (fin, EW), 1)
    ml = (c == j).astype(jnp.float32) * (j < heads).astype(jnp.float32)
    mr = (c == (j - 4)).astype(jnp.float32) * ((j - 4) >= 0).astype(jnp.float32) \
        * ((j - 4) < heads).astype(jnp.float32)
    return ml, mr


def _sentinel(heads):
    lane = lax.broadcasted_iota(jnp.int32, (NN, EW), 1)
    return (lane < heads).astype(jnp.float32)


def _prep0_body(x_ref, w_ref, al_ref, ar_ref, feat_ref, eler_ref):
    feat = jnp.dot(x_ref[...], w_ref[...], preferred_element_type=jnp.float32)
    feat_ref[...] = jnp.concatenate([feat, _sentinel(4)], axis=1)
    ml, mr = _head_masks(256, 4, 64)
    eler_ref[...] = (
        jnp.dot(feat * al_ref[...], ml, preferred_element_type=jnp.float32)
        + jnp.dot(feat * ar_ref[...], mr, preferred_element_type=jnp.float32))


def _tc_prep0(x, w0, al_row, ar_row):
    return pl.pallas_call(
        _prep0_body,
        out_shape=(jax.ShapeDtypeStruct((NN, 256 + EW), jnp.float32),
                   jax.ShapeDtypeStruct((NN, EW), jnp.float32)),
    )(x, w0, al_row, ar_row)


def _esum_bcast4(es):
    return jnp.concatenate(
        [jnp.broadcast_to(es[:, h:h + 1], (NN, 64)) for h in range(4)], axis=1)


def _post_prep_body(acc_ref, res_ref, b_ref, w_ref, al_ref, ar_ref,
                    h_ref, feat_ref, eler_ref, *, identity_res, fout, heads,
                    hd, next_heads):
    acc = acc_ref[0] + acc_ref[1]
    es = acc[:, 256:256 + 4]
    rst = acc[:, 0:256] / (_esum_bcast4(es) + 1e-9)
    if identity_res:
        rst = rst + res_ref[...]
    rst = rst + b_ref[...]
    h = jnp.where(rst > 0, rst, jnp.exp(rst) - 1.0)
    h_ref[...] = h
    feat = jnp.dot(h, w_ref[...], preferred_element_type=jnp.float32)
    feat_ref[...] = jnp.concatenate([feat, _sentinel(next_heads)], axis=1)
    ml, mr = _head_masks(fout, heads, hd)
    eler_ref[...] = (
        jnp.dot(feat * al_ref[...], ml, preferred_element_type=jnp.float32)
        + jnp.dot(feat * ar_ref[...], mr, preferred_element_type=jnp.float32))


def _tc_post_prep(acc, res, b_row, w, al_row, ar_row, identity_res, fout,
                  heads, hd, next_heads):
    body = functools.partial(_post_prep_body, identity_res=identity_res,
                             fout=fout, heads=heads, hd=hd,
                             next_heads=next_heads)
    return pl.pallas_call(
        body,
        out_shape=(jax.ShapeDtypeStruct((NN, 256), jnp.float32),
                   jax.ShapeDtypeStruct((NN, fout + EW), jnp.float32),
                   jax.ShapeDtypeStruct((NN, EW), jnp.float32)),
    )(acc, res, b_row, w, al_row, ar_row)


def _emb_body(acc_ref, h_ref, rw_ref, b_ref, emb_ref):
    acc = acc_ref[0] + acc_ref[1]
    es = acc[:, EW:EW + 1]
    emb_ref[...] = (acc[:, 0:EW] / (es + 1e-9)
                    + jnp.dot(h_ref[...], rw_ref[...],
                              preferred_element_type=jnp.float32)
                    + b_ref[...])


def _tc_emb(acc, h2, rw_pad, b_row):
    return pl.pallas_call(
        _emb_body,
        out_shape=jax.ShapeDtypeStruct((NN, EW), jnp.float32),
    )(acc, h2, rw_pad, b_row)


def _grav_body(emb_ref, embt_ref, dist_ref, scal_ref, out_ref):
    i0 = pl.program_id(0) * 128
    acc = jnp.zeros((128, NN), jnp.float32)
    for k in range(1, 5):
        d = embt_ref[k:k + 1, :] - emb_ref[:, k:k + 1]
        acc = acc + d * d
    lam1 = scal_ref[0]
    lam2 = scal_ref[1]
    lam3 = scal_ref[2]
    beta = scal_ref[3]
    g = scal_ref[4]
    p1 = jnp.power(jnp.abs(embt_ref[0:1, :]), lam1)
    p2 = jnp.power(jnp.abs(emb_ref[:, 0:1]), lam2) * g
    x = jnp.maximum(acc * 0.25 + beta * dist_ref[...], 1e-7)
    od = p1 * p2 * jnp.exp(jnp.log(x) * (-0.5 * lam3))
    row = lax.broadcasted_iota(jnp.int32, (128, NN), 0) + i0
    col = lax.broadcasted_iota(jnp.int32, (128, NN), 1)
    out_ref[...] = jnp.where(row == col, 0.0, od)


def _tc_grav(emb, embt, distance, scal):
    return pl.pallas_call(
        _grav_body,
        grid=(NN // 128,),
        in_specs=[
            pl.BlockSpec((128, EW), lambda i: (i, 0)),
            pl.BlockSpec((EW, NN), lambda i: (0, 0)),
            pl.BlockSpec((128, NN), lambda i: (i, 0)),
            pl.BlockSpec(memory_space=pltpu.SMEM),
        ],
        out_specs=pl.BlockSpec((128, NN), lambda i: (i, 0)),
        out_shape=jax.ShapeDtypeStruct((NN, NN), jnp.float32),
    )(emb, embt, distance, scal)


def _edge_sc_call(fw, heads):
    hseg = (fw - EW) // (heads * 16)
    mesh = plsc.VectorSubcoreMesh(core_axis_name="c", subcore_axis_name="s")
    out_type = jax.ShapeDtypeStruct((2, NN, fw), jnp.float32)
    scratch = [
        pltpu.VMEM((NCH, CH), jnp.int32),
        pltpu.VMEM((NCH, CH), jnp.int32),
        pltpu.VMEM((NN * EW,), jnp.float32),
        [pltpu.VMEM((CH, fw), jnp.float32) for _ in range(NBUF)],
        pltpu.VMEM((CH * EW,), jnp.float32),
        pltpu.VMEM_SHARED((NN, fw), jnp.float32),
        [pltpu.SemaphoreType.DMA for _ in range(NBUF)],
        [pltpu.SemaphoreType.DMA for _ in range(NBUF)],
    ]

    @functools.partial(
        pl.kernel, mesh=mesh, out_type=out_type, scratch_types=scratch,
        compiler_params=pltpu.CompilerParams(needs_layout_passes=False,
                                             use_tc_tiling_on_sc=False))
    def k(src_hbm, dst_hbm, eler_hbm, feat_hbm, acc_out,
          src_v, dst_v, eler_v, rows, w_v, acc_sp, gsem, ssem):
        cid = lax.axis_index("c")
        sid = lax.axis_index("s")
        wid = cid * 16 + sid
        pltpu.sync_copy(src_hbm.at[pl.ds(wid * NCH, NCH)], src_v)
        pltpu.sync_copy(dst_hbm.at[pl.ds(wid * NCH, NCH)], dst_v)
        pltpu.sync_copy(eler_hbm, eler_v)

        zero16 = jnp.zeros((16,), jnp.float32)
        gathers = {}
        scatters = {}

        def fire_gather(c):
            gathers[c] = pltpu.async_copy(
                feat_hbm.at[src_v.at[c]], rows[c % NBUF], gsem[c % NBUF])

        for c in range(NBUF - 1):
            fire_gather(c)

        def zw(i, _):
            w_v[pl.ds(i * 16, 16)] = zero16
            return 0

        lax.fori_loop(0, CH, zw, 0)

        def zrow(i, _):
            for s in range(fw // 16):
                rows[NBUF - 1][i, pl.ds(s * 16, 16)] = zero16
            return 0

        lax.fori_loop(0, CH, zrow, 0)
        for piece in range(128 // CH):
            pltpu.sync_copy(rows[NBUF - 1],
                            acc_sp.at[pl.ds(sid * 128 + piece * CH, CH)])

        plsc.subcore_barrier()

        ones16 = jnp.full((16,), 1, jnp.int32)
        for c in range(NCH):
            b = c % NBUF
            for g in range(CH // 16):
                sidx = src_v[c, pl.ds(g * 16, 16)]
                didx = dst_v[c, pl.ds(g * 16, 16)]
                rowi = (lax.iota(jnp.int32, 16) + g * 16) * EW
                for h in range(heads):
                    el = plsc.load_gather(eler_v, [sidx * EW + h])
                    er = plsc.load_gather(eler_v, [didx * EW + (4 + h)])
                    e = el + er
                    w = jnp.exp(jnp.maximum(e, NEG * e))
                    plsc.store_scatter(w_v, [rowi + h], w)
            gathers[c].wait()

            def scale_edge(i, _):
                off = i * EW
                wvec = w_v[pl.ds(off, 16)]
                for h in range(heads):
                    wh = plsc.load_gather(w_v, [ones16 * (off + h)])
                    for s in range(hseg):
                        sl = pl.ds((h * hseg + s) * 16, 16)
                        rows[b][i, sl] = rows[b][i, sl] * wh
                sl = pl.ds(fw - EW, 16)
                rows[b][i, sl] = rows[b][i, sl] * wvec
                return 0

            lax.fori_loop(0, CH, scale_edge, 0)
            scatters[c] = pltpu.async_copy(
                rows[b], acc_sp.at[dst_v.at[c]], ssem[b], add=True)
            nxt = c + NBUF - 1
            if nxt < NCH:
                if nxt >= NBUF:
                    scatters[nxt - NBUF].wait()
                fire_gather(nxt)
        for c in range(NCH - NBUF, NCH):
            scatters[c].wait()

        plsc.subcore_barrier()
        for piece in range(128 // CH):
            sl = pl.ds(sid * 128 + piece * CH, CH)
            pltpu.sync_copy(acc_sp.at[sl], rows[piece % NBUF])
            pltpu.sync_copy(rows[piece % NBUF], acc_out.at[cid, sl])

    return k


def _edge_pass(src2d, dst2d, eler_flat, feat, fw, heads):
    return _edge_sc_call(fw, heads)(src2d, dst2d, eler_flat, feat)


def kernel(region_attributes, distance, edge_index, W0, al0, ar0, b0,
           W1, al1, ar1, b1, W2, al2, ar2, b2, resW2,
           lam1, lam2, lam3, beta, G):
    x = region_attributes
    src2d = edge_index[0].astype(jnp.int32).reshape(EE // CH, CH)
    dst2d = edge_index[1].astype(jnp.int32).reshape(EE // CH, CH)

    al0r = al0.reshape(1, 256)
    ar0r = ar0.reshape(1, 256)
    al1r = al1.reshape(1, 256)
    ar1r = ar1.reshape(1, 256)
    pad5 = lambda a: jnp.pad(a.reshape(1, 5), ((0, 0), (0, EW - 5)))
    al2r = pad5(al2)
    ar2r = pad5(ar2)
    b2r = pad5(b2)
    w2p = jnp.pad(W2, ((0, 0), (0, EW - 5)))
    rw2p = jnp.pad(resW2, ((0, 0), (0, EW - 5)))

    feat0, eler0 = _tc_prep0(x, W0, al0r, ar0r)
    acc0 = _edge_pass(src2d, dst2d, eler0.reshape(-1), feat0, 256 + EW, 4)
    h1, feat1, eler1 = _tc_post_prep(acc0, feat0[:, 0:256], b0.reshape(1, 256),
                                     W1, al1r, ar1r, False, 256, 4, 64, 4)
    acc1 = _edge_pass(src2d, dst2d, eler1.reshape(-1), feat1, 256 + EW, 4)
    h2, feat2, eler2 = _tc_post_prep(acc1, h1, b1.reshape(1, 256),
                                     w2p, al2r, ar2r, True, EW, 1, 5, 1)
    acc2 = _edge_pass(src2d, dst2d, eler2.reshape(-1), feat2, EW + EW, 1)
    emb = _tc_emb(acc2, h2, rw2p, b2r)
    embt = emb.T
    scal = jnp.concatenate([lam1, lam2, lam3, beta, G]).astype(jnp.float32)
    return _tc_grav(emb, embt, distance, scal)


# speedup vs baseline: 35.4238x; 1.0825x over previous
import functools
import jax
import jax.numpy as jnp
from jax import lax
from jax.experimental import pallas as pl
from jax.experimental.pallas import tpu as pltpu
from jax.experimental.pallas import tpu_sc as plsc

NN = 2048
EE = 32768
NEG = 0.2
EW = 16
CH = 64
NWORK = 32
NCH = EE // (NWORK * CH)
NBUF = 3


def _head_masks(fin, heads, hd):
    c = lax.broadcasted_iota(jnp.int32, (fin, EW), 0) // hd
    j = lax.broadcasted_iota(jnp.int32, (fin, EW), 1)
    ml = (c == j).astype(jnp.float32) * (j < heads).astype(jnp.float32)
    mr = (c == (j - 4)).astype(jnp.float32) * ((j - 4) >= 0).astype(jnp.float32) \
        * ((j - 4) < heads).astype(jnp.float32)
    return ml, mr


def _sentinel(heads):
    lane = lax.broadcasted_iota(jnp.int32, (NN, EW), 1)
    return (lane < heads).astype(jnp.float32)


def _prep0_body(x_ref, w_ref, al_ref, ar_ref, feat_ref, eler_ref):
    feat = jnp.dot(x_ref[...], w_ref[...], preferred_element_type=jnp.float32)
    feat_ref[...] = jnp.concatenate([feat, _sentinel(4)], axis=1)
    ml, mr = _head_masks(256, 4, 64)
    eler_ref[...] = (
        jnp.dot(feat * al_ref[...], ml, preferred_element_type=jnp.float32)
        + jnp.dot(feat * ar_ref[...], mr, preferred_element_type=jnp.float32))


def _tc_prep0(x, w0, al_row, ar_row):
    return pl.pallas_call(
        _prep0_body,
        out_shape=(jax.ShapeDtypeStruct((NN, 256 + EW), jnp.float32),
                   jax.ShapeDtypeStruct((NN, EW), jnp.float32)),
    )(x, w0, al_row, ar_row)


def _esum_bcast4(es):
    return jnp.concatenate(
        [jnp.broadcast_to(es[:, h:h + 1], (NN, 64)) for h in range(4)], axis=1)


def _post_prep_body(acc_ref, res_ref, b_ref, w_ref, al_ref, ar_ref,
                    h_ref, feat_ref, eler_ref, *, identity_res, fout, heads,
                    hd, next_heads):
    acc = acc_ref[0] + acc_ref[1]
    es = acc[:, 256:256 + 4]
    rst = acc[:, 0:256] / (_esum_bcast4(es) + 1e-9)
    if identity_res:
        rst = rst + res_ref[...]
    rst = rst + b_ref[...]
    h = jnp.where(rst > 0, rst, jnp.exp(rst) - 1.0)
    h_ref[...] = h
    feat = jnp.dot(h, w_ref[...], preferred_element_type=jnp.float32)
    feat_ref[...] = jnp.concatenate([feat, _sentinel(next_heads)], axis=1)
    ml, mr = _head_masks(fout, heads, hd)
    eler_ref[...] = (
        jnp.dot(feat * al_ref[...], ml, preferred_element_type=jnp.float32)
        + jnp.dot(feat * ar_ref[...], mr, preferred_element_type=jnp.float32))


def _tc_post_prep(acc, res, b_row, w, al_row, ar_row, identity_res, fout,
                  heads, hd, next_heads):
    body = functools.partial(_post_prep_body, identity_res=identity_res,
                             fout=fout, heads=heads, hd=hd,
                             next_heads=next_heads)
    return pl.pallas_call(
        body,
        out_shape=(jax.ShapeDtypeStruct((NN, 256), jnp.float32),
                   jax.ShapeDtypeStruct((NN, fout + EW), jnp.float32),
                   jax.ShapeDtypeStruct((NN, EW), jnp.float32)),
    )(acc, res, b_row, w, al_row, ar_row)


def _emb_body(acc_ref, h_ref, rw_ref, b_ref, emb_ref):
    acc = acc_ref[0] + acc_ref[1]
    es = acc[:, EW:EW + 1]
    emb_ref[...] = (acc[:, 0:EW] / (es + 1e-9)
                    + jnp.dot(h_ref[...], rw_ref[...],
                              preferred_element_type=jnp.float32)
                    + b_ref[...])


def _tc_emb(acc, h2, rw_pad, b_row):
    return pl.pallas_call(
        _emb_body,
        out_shape=jax.ShapeDtypeStruct((NN, EW), jnp.float32),
    )(acc, h2, rw_pad, b_row)


def _grav_body(emb_ref, embt_ref, dist_ref, scal_ref, out_ref):
    i0 = pl.program_id(0) * 128
    acc = jnp.zeros((128, NN), jnp.float32)
    for k in range(1, 5):
        d = embt_ref[k:k + 1, :] - emb_ref[:, k:k + 1]
        acc = acc + d * d
    lam1 = scal_ref[0]
    lam2 = scal_ref[1]
    lam3 = scal_ref[2]
    beta = scal_ref[3]
    g = scal_ref[4]
    p1 = jnp.power(jnp.abs(embt_ref[0:1, :]), lam1)
    p2 = jnp.power(jnp.abs(emb_ref[:, 0:1]), lam2) * g
    x = jnp.maximum(acc * 0.25 + beta * dist_ref[...], 1e-7)
    od = p1 * p2 * jnp.exp(jnp.log(x) * (-0.5 * lam3))
    row = lax.broadcasted_iota(jnp.int32, (128, NN), 0) + i0
    col = lax.broadcasted_iota(jnp.int32, (128, NN), 1)
    out_ref[...] = jnp.where(row == col, 0.0, od)


def _tc_grav(emb, embt, distance, scal):
    return pl.pallas_call(
        _grav_body,
        grid=(NN // 128,),
        in_specs=[
            pl.BlockSpec((128, EW), lambda i: (i, 0)),
            pl.BlockSpec((EW, NN), lambda i: (0, 0)),
            pl.BlockSpec((128, NN), lambda i: (i, 0)),
            pl.BlockSpec(memory_space=pltpu.SMEM),
        ],
        out_specs=pl.BlockSpec((128, NN), lambda i: (i, 0)),
        out_shape=jax.ShapeDtypeStruct((NN, NN), jnp.float32),
    )(emb, embt, distance, scal)


def _edge_sc_call(fw, heads):
    hseg = (fw - EW) // (heads * 16)
    mesh = plsc.VectorSubcoreMesh(core_axis_name="c", subcore_axis_name="s")
    out_type = jax.ShapeDtypeStruct((2, NN, fw), jnp.float32)
    scratch = [
        pltpu.VMEM((NCH, CH), jnp.int32),
        pltpu.VMEM((NCH, CH), jnp.int32),
        pltpu.VMEM((NN, EW), jnp.float32),
        [pltpu.VMEM((CH, fw), jnp.float32) for _ in range(NBUF)],
        pltpu.VMEM((CH * EW,), jnp.float32),
        pltpu.VMEM_SHARED((NN, fw), jnp.float32),
        [pltpu.SemaphoreType.DMA for _ in range(NBUF)],
        [pltpu.SemaphoreType.DMA for _ in range(NBUF)],
    ]

    @functools.partial(
        pl.kernel, mesh=mesh, out_type=out_type, scratch_types=scratch,
        compiler_params=pltpu.CompilerParams(needs_layout_passes=False,
                                             use_tc_tiling_on_sc=False))
    def k(src_hbm, dst_hbm, eler_hbm, feat_hbm, acc_out,
          src_v, dst_v, eler_v, rows, w_v, acc_sp, gsem, ssem):
        cid = lax.axis_index("c")
        sid = lax.axis_index("s")
        wid = cid * 16 + sid
        pltpu.sync_copy(src_hbm.at[pl.ds(wid * NCH, NCH)], src_v)
        pltpu.sync_copy(dst_hbm.at[pl.ds(wid * NCH, NCH)], dst_v)
        pltpu.sync_copy(eler_hbm, eler_v)

        zero16 = jnp.zeros((16,), jnp.float32)
        gathers = {}
        scatters = {}

        def fire_gather(c):
            gathers[c] = pltpu.async_copy(
                feat_hbm.at[src_v.at[c]], rows[c % NBUF], gsem[c % NBUF])

        for c in range(NBUF - 1):
            fire_gather(c)

        def zw(i, _):
            w_v[pl.ds(i * 16, 16)] = zero16
            return 0

        lax.fori_loop(0, CH, zw, 0)

        def zrow(i, _):
            for s in range(fw // 16):
                rows[NBUF - 1][i, pl.ds(s * 16, 16)] = zero16
            return 0

        lax.fori_loop(0, CH, zrow, 0)
        for piece in range(128 // CH):
            pltpu.sync_copy(rows[NBUF - 1],
                            acc_sp.at[pl.ds(sid * 128 + piece * CH, CH)])

        plsc.subcore_barrier()

        ones16 = jnp.full((16,), 1, jnp.int32)
        for c in range(NCH):
            b = c % NBUF
            for g in range(CH // 16):
                sidx = src_v[c, pl.ds(g * 16, 16)]
                didx = dst_v[c, pl.ds(g * 16, 16)]
                rowi = (lax.iota(jnp.int32, 16) + g * 16) * EW
                for h in range(heads):
                    el = plsc.load_gather(
                        eler_v, [sidx, jnp.full((16,), h, jnp.int32)])
                    er = plsc.load_gather(
                        eler_v, [didx, jnp.full((16,), 4 + h, jnp.int32)])
                    e = el + er
                    w = jnp.exp(jnp.maximum(e, NEG * e))
                    plsc.store_scatter(w_v, [rowi + h], w)
            gathers[c].wait()

            def scale_edge(i, rows_b=rows[b]):
                off = i * EW
                wvec = w_v[pl.ds(off, 16)]
                for h in range(heads):
                    wh = plsc.load_gather(w_v, [ones16 * (off + h)])
                    for s in range(hseg):
                        sl = pl.ds((h * hseg + s) * 16, 16)
                        rows_b[i, sl] = rows_b[i, sl] * wh
                sl = pl.ds(fw - EW, 16)
                rows_b[i, sl] = rows_b[i, sl] * wvec

            plsc.parallel_loop(0, CH, 1, unroll=4)(scale_edge)
            scatters[c] = pltpu.async_copy(
                rows[b], acc_sp.at[dst_v.at[c]], ssem[b], add=True)
            nxt = c + NBUF - 1
            if nxt < NCH:
                if nxt >= NBUF:
                    scatters[nxt - NBUF].wait()
                fire_gather(nxt)
        for c in range(NCH - NBUF, NCH):
            scatters[c].wait()

        plsc.subcore_barrier()
        for piece in range(128 // CH):
            sl = pl.ds(sid * 128 + piece * CH, CH)
            pltpu.sync_copy(acc_sp.at[sl], rows[piece % NBUF])
            pltpu.sync_copy(rows[piece % NBUF], acc_out.at[cid, sl])

    return k


def _edge_pass(src2d, dst2d, eler_flat, feat, fw, heads):
    return _edge_sc_call(fw, heads)(src2d, dst2d, eler_flat, feat)


def kernel(region_attributes, distance, edge_index, W0, al0, ar0, b0,
           W1, al1, ar1, b1, W2, al2, ar2, b2, resW2,
           lam1, lam2, lam3, beta, G):
    x = region_attributes
    src2d = edge_index[0].astype(jnp.int32).reshape(EE // CH, CH)
    dst2d = edge_index[1].astype(jnp.int32).reshape(EE // CH, CH)

    al0r = al0.reshape(1, 256)
    ar0r = ar0.reshape(1, 256)
    al1r = al1.reshape(1, 256)
    ar1r = ar1.reshape(1, 256)
    pad5 = lambda a: jnp.pad(a.reshape(1, 5), ((0, 0), (0, EW - 5)))
    al2r = pad5(al2)
    ar2r = pad5(ar2)
    b2r = pad5(b2)
    w2p = jnp.pad(W2, ((0, 0), (0, EW - 5)))
    rw2p = jnp.pad(resW2, ((0, 0), (0, EW - 5)))

    feat0, eler0 = _tc_prep0(x, W0, al0r, ar0r)
    acc0 = _edge_pass(src2d, dst2d, eler0, feat0, 256 + EW, 4)
    h1, feat1, eler1 = _tc_post_prep(acc0, feat0[:, 0:256], b0.reshape(1, 256),
                                     W1, al1r, ar1r, False, 256, 4, 64, 4)
    acc1 = _edge_pass(src2d, dst2d, eler1, feat1, 256 + EW, 4)
    h2, feat2, eler2 = _tc_post_prep(acc1, h1, b1.reshape(1, 256),
                                     w2p, al2r, ar2r, True, EW, 1, 5, 1)
    acc2 = _edge_pass(src2d, dst2d, eler2, feat2, EW + EW, 1)
    emb = _tc_emb(acc2, h2, rw2p, b2r)
    embt = emb.T
    scal = jnp.concatenate([lam1, lam2, lam3, beta, G]).astype(jnp.float32)
    return _tc_grav(emb, embt, distance, scal)


# speedup vs baseline: 36.0030x; 1.0164x over previous
import functools
import jax
import jax.numpy as jnp
from jax import lax
from jax.experimental import pallas as pl
from jax.experimental.pallas import tpu as pltpu
from jax.experimental.pallas import tpu_sc as plsc

NN = 2048
EE = 32768
NEG = 0.2
EW = 16
CH = 64
NWORK = 32
NCH = EE // (NWORK * CH)
NBUF = 3


def _head_masks(fin, heads, hd):
    c = lax.broadcasted_iota(jnp.int32, (fin, EW), 0) // hd
    j = lax.broadcasted_iota(jnp.int32, (fin, EW), 1)
    ml = (c == j).astype(jnp.float32) * (j < heads).astype(jnp.float32)
    mr = (c == (j - 4)).astype(jnp.float32) * ((j - 4) >= 0).astype(jnp.float32) \
        * ((j - 4) < heads).astype(jnp.float32)
    return ml, mr


def _sentinel(heads):
    lane = lax.broadcasted_iota(jnp.int32, (NN, EW), 1)
    return (lane < heads).astype(jnp.float32)


def _prep0_body(x_ref, w_ref, al_ref, ar_ref, feat_ref, eler_ref):
    feat = jnp.dot(x_ref[...], w_ref[...], preferred_element_type=jnp.float32)
    feat_ref[...] = jnp.concatenate([feat, _sentinel(4)], axis=1)
    ml, mr = _head_masks(256, 4, 64)
    eler_ref[...] = (
        jnp.dot(feat * al_ref[...], ml, preferred_element_type=jnp.float32)
        + jnp.dot(feat * ar_ref[...], mr, preferred_element_type=jnp.float32))


def _tc_prep0(x, w0, al_row, ar_row):
    return pl.pallas_call(
        _prep0_body,
        out_shape=(jax.ShapeDtypeStruct((NN, 256 + EW), jnp.float32),
                   jax.ShapeDtypeStruct((NN, EW), jnp.float32)),
    )(x, w0, al_row, ar_row)


def _esum_bcast4(es):
    return jnp.concatenate(
        [jnp.broadcast_to(es[:, h:h + 1], (NN, 64)) for h in range(4)], axis=1)


def _post_prep_body(acc_ref, res_ref, b_ref, w_ref, al_ref, ar_ref,
                    h_ref, feat_ref, eler_ref, *, identity_res, fout, heads,
                    hd, next_heads):
    acc = acc_ref[0] + acc_ref[1]
    es = acc[:, 256:256 + 4]
    rst = acc[:, 0:256] / (_esum_bcast4(es) + 1e-9)
    if identity_res:
        rst = rst + res_ref[...]
    rst = rst + b_ref[...]
    h = jnp.where(rst > 0, rst, jnp.exp(rst) - 1.0)
    h_ref[...] = h
    feat = jnp.dot(h, w_ref[...], preferred_element_type=jnp.float32)
    feat_ref[...] = jnp.concatenate([feat, _sentinel(next_heads)], axis=1)
    ml, mr = _head_masks(fout, heads, hd)
    eler_ref[...] = (
        jnp.dot(feat * al_ref[...], ml, preferred_element_type=jnp.float32)
        + jnp.dot(feat * ar_ref[...], mr, preferred_element_type=jnp.float32))


def _tc_post_prep(acc, res, b_row, w, al_row, ar_row, identity_res, fout,
                  heads, hd, next_heads):
    body = functools.partial(_post_prep_body, identity_res=identity_res,
                             fout=fout, heads=heads, hd=hd,
                             next_heads=next_heads)
    return pl.pallas_call(
        body,
        out_shape=(jax.ShapeDtypeStruct((NN, 256), jnp.float32),
                   jax.ShapeDtypeStruct((NN, fout + EW), jnp.float32),
                   jax.ShapeDtypeStruct((NN, EW), jnp.float32)),
    )(acc, res, b_row, w, al_row, ar_row)


def _grav_body(accs_ref, h_ref, rw_ref, b_ref, dist_ref, scal_ref, out_ref,
               emb_sc, embt_sc):
    i = pl.program_id(0)

    @pl.when(i == 0)
    def _init():
        accs = accs_ref[0] + accs_ref[1]
        es = accs[:, EW:EW + 1]
        emb = (accs[:, 0:EW] / (es + 1e-9)
               + jnp.dot(h_ref[...], rw_ref[...],
                         preferred_element_type=jnp.float32)
               + b_ref[...])
        emb_sc[...] = emb
        n_ = lax.broadcasted_iota(jnp.int32, (NN, NN), 0)
        j_ = lax.broadcasted_iota(jnp.int32, (NN, NN), 1)
        eye = (n_ == j_).astype(jnp.float32)
        embt_sc[...] = jax.lax.dot_general(
            emb, eye, (((0,), (0,)), ((), ())),
            preferred_element_type=jnp.float32)

    i0 = i * 128
    emb_ref = emb_sc[pl.ds(i0, 128), :]
    embt_ref = embt_sc
    acc = jnp.zeros((128, NN), jnp.float32)
    for k in range(1, 5):
        d = embt_ref[k:k + 1, :] - emb_ref[:, k:k + 1]
        acc = acc + d * d
    lam1 = scal_ref[0]
    lam2 = scal_ref[1]
    lam3 = scal_ref[2]
    beta = scal_ref[3]
    g = scal_ref[4]
    p1 = jnp.power(jnp.abs(embt_ref[0:1, :]), lam1)
    p2 = jnp.power(jnp.abs(emb_ref[:, 0:1]), lam2) * g
    x = jnp.maximum(acc * 0.25 + beta * dist_ref[...], 1e-7)
    od = p1 * p2 * jnp.exp(jnp.log(x) * (-0.5 * lam3))
    row = lax.broadcasted_iota(jnp.int32, (128, NN), 0) + i0
    col = lax.broadcasted_iota(jnp.int32, (128, NN), 1)
    out_ref[...] = jnp.where(row == col, 0.0, od)


def _tc_grav(accs, h2, rw_pad, b_row, distance, scal):
    return pl.pallas_call(
        _grav_body,
        grid=(NN // 128,),
        in_specs=[
            pl.BlockSpec((2, NN, 2 * EW), lambda i: (0, 0, 0)),
            pl.BlockSpec((NN, 256), lambda i: (0, 0)),
            pl.BlockSpec((256, EW), lambda i: (0, 0)),
            pl.BlockSpec((1, EW), lambda i: (0, 0)),
            pl.BlockSpec((128, NN), lambda i: (i, 0)),
            pl.BlockSpec(memory_space=pltpu.SMEM),
        ],
        out_specs=pl.BlockSpec((128, NN), lambda i: (i, 0)),
        out_shape=jax.ShapeDtypeStruct((NN, NN), jnp.float32),
        scratch_shapes=[pltpu.VMEM((NN, EW), jnp.float32),
                        pltpu.VMEM((EW, NN), jnp.float32)],
    )(accs, h2, rw_pad, b_row, distance, scal)


def _edge_sc_call(fw, heads):
    hseg = (fw - EW) // (heads * 16)
    mesh = plsc.VectorSubcoreMesh(core_axis_name="c", subcore_axis_name="s")
    out_type = jax.ShapeDtypeStruct((2, NN, fw), jnp.float32)
    scratch = [
        pltpu.VMEM((NCH, CH), jnp.int32),
        pltpu.VMEM((NCH, CH), jnp.int32),
        pltpu.VMEM((NN, EW), jnp.float32),
        [pltpu.VMEM((CH, fw), jnp.float32) for _ in range(NBUF)],
        pltpu.VMEM((CH * EW,), jnp.float32),
        pltpu.VMEM_SHARED((NN, fw), jnp.float32),
        [pltpu.SemaphoreType.DMA for _ in range(NBUF)],
        [pltpu.SemaphoreType.DMA for _ in range(NBUF)],
    ]

    @functools.partial(
        pl.kernel, mesh=mesh, out_type=out_type, scratch_types=scratch,
        compiler_params=pltpu.CompilerParams(needs_layout_passes=False,
                                             use_tc_tiling_on_sc=False))
    def k(src_hbm, dst_hbm, eler_hbm, feat_hbm, acc_out,
          src_v, dst_v, eler_v, rows, w_v, acc_sp, gsem, ssem):
        cid = lax.axis_index("c")
        sid = lax.axis_index("s")
        wid = cid * 16 + sid
        pltpu.sync_copy(src_hbm.at[pl.ds(wid * NCH, NCH)], src_v)
        pltpu.sync_copy(dst_hbm.at[pl.ds(wid * NCH, NCH)], dst_v)
        pltpu.sync_copy(eler_hbm, eler_v)

        zero16 = jnp.zeros((16,), jnp.float32)
        gathers = {}
        scatters = {}

        def fire_gather(c):
            gathers[c] = pltpu.async_copy(
                feat_hbm.at[src_v.at[c]], rows[c % NBUF], gsem[c % NBUF])

        for c in range(NBUF - 1):
            fire_gather(c)

        def zw(i, _):
            w_v[pl.ds(i * 16, 16)] = zero16
            return 0

        lax.fori_loop(0, CH, zw, 0)

        def zrow(i, _):
            for s in range(fw // 16):
                rows[NBUF - 1][i, pl.ds(s * 16, 16)] = zero16
            return 0

        lax.fori_loop(0, CH, zrow, 0)
        for piece in range(128 // CH):
            pltpu.sync_copy(rows[NBUF - 1],
                            acc_sp.at[pl.ds(sid * 128 + piece * CH, CH)])

        plsc.subcore_barrier()

        ones16 = jnp.full((16,), 1, jnp.int32)
        for c in range(NCH):
            b = c % NBUF
            for g in range(CH // 16):
                sidx = src_v[c, pl.ds(g * 16, 16)]
                didx = dst_v[c, pl.ds(g * 16, 16)]
                rowi = (lax.iota(jnp.int32, 16) + g * 16) * EW
                for h in range(heads):
                    el = plsc.load_gather(
                        eler_v, [sidx, jnp.full((16,), h, jnp.int32)])
                    er = plsc.load_gather(
                        eler_v, [didx, jnp.full((16,), 4 + h, jnp.int32)])
                    e = el + er
                    w = jnp.exp(jnp.maximum(e, NEG * e))
                    plsc.store_scatter(w_v, [rowi + h], w)
            gathers[c].wait()

            def scale_edge(i, rows_b=rows[b]):
                off = i * EW
                wvec = w_v[pl.ds(off, 16)]
                for h in range(heads):
                    wh = plsc.load_gather(w_v, [ones16 * (off + h)])
                    for s in range(hseg):
                        sl = pl.ds((h * hseg + s) * 16, 16)
                        rows_b[i, sl] = rows_b[i, sl] * wh
                sl = pl.ds(fw - EW, 16)
                rows_b[i, sl] = rows_b[i, sl] * wvec

            plsc.parallel_loop(0, CH, 1, unroll=4)(scale_edge)
            scatters[c] = pltpu.async_copy(
                rows[b], acc_sp.at[dst_v.at[c]], ssem[b], add=True)
            nxt = c + NBUF - 1
            if nxt < NCH:
                if nxt >= NBUF:
                    scatters[nxt - NBUF].wait()
                fire_gather(nxt)
        for c in range(NCH - NBUF, NCH):
            scatters[c].wait()

        plsc.subcore_barrier()
        for piece in range(128 // CH):
            sl = pl.ds(sid * 128 + piece * CH, CH)
            pltpu.sync_copy(acc_sp.at[sl], rows[piece % NBUF])
            pltpu.sync_copy(rows[piece % NBUF], acc_out.at[cid, sl])

    return k


def _edge_pass(src2d, dst2d, eler_flat, feat, fw, heads):
    return _edge_sc_call(fw, heads)(src2d, dst2d, eler_flat, feat)


def kernel(region_attributes, distance, edge_index, W0, al0, ar0, b0,
           W1, al1, ar1, b1, W2, al2, ar2, b2, resW2,
           lam1, lam2, lam3, beta, G):
    x = region_attributes
    src2d = edge_index[0].astype(jnp.int32).reshape(EE // CH, CH)
    dst2d = edge_index[1].astype(jnp.int32).reshape(EE // CH, CH)

    al0r = al0.reshape(1, 256)
    ar0r = ar0.reshape(1, 256)
    al1r = al1.reshape(1, 256)
    ar1r = ar1.reshape(1, 256)
    pad5 = lambda a: jnp.pad(a.reshape(1, 5), ((0, 0), (0, EW - 5)))
    al2r = pad5(al2)
    ar2r = pad5(ar2)
    b2r = pad5(b2)
    w2p = jnp.pad(W2, ((0, 0), (0, EW - 5)))
    rw2p = jnp.pad(resW2, ((0, 0), (0, EW - 5)))

    feat0, eler0 = _tc_prep0(x, W0, al0r, ar0r)
    acc0 = _edge_pass(src2d, dst2d, eler0, feat0, 256 + EW, 4)
    h1, feat1, eler1 = _tc_post_prep(acc0, feat0[:, 0:256], b0.reshape(1, 256),
                                     W1, al1r, ar1r, False, 256, 4, 64, 4)
    acc1 = _edge_pass(src2d, dst2d, eler1, feat1, 256 + EW, 4)
    h2, feat2, eler2 = _tc_post_prep(acc1, h1, b1.reshape(1, 256),
                                     w2p, al2r, ar2r, True, EW, 1, 5, 1)
    acc2 = _edge_pass(src2d, dst2d, eler2, feat2, EW + EW, 1)
    scal = jnp.concatenate([lam1, lam2, lam3, beta, G]).astype(jnp.float32)
    return _tc_grav(acc2, h2, rw2p, b2r, distance, scal)
